# baseline (speedup 1.0000x reference)
import functools

import jax
import jax.numpy as jnp
from jax.experimental import pallas as pl
from jax.experimental.pallas import tpu as pltpu

_BM = 200
_NBUF = 4


def _gcn_kernel(x_ref, w1_ref, w2_ref, adj_ref, out_ref,
                abuf, s1_ref, s2_ref, sem, *, nb):
    p = pl.program_id(0)
    b = pl.program_id(1)
    t = p * nb + b
    n = adj_ref.shape[1]

    def _start(tp):
        r = jax.lax.rem(tp, nb)
        slot = jax.lax.rem(tp, _NBUF)
        pltpu.make_async_copy(
            adj_ref.at[pl.ds(r * _BM, _BM), :],
            abuf.at[slot],
            sem.at[slot],
        ).start()

    @pl.when(t == 0)
    def _prime():
        for tp in range(_NBUF):
            _start(tp)
        s1_ref[...] = jnp.dot(x_ref[...], w1_ref[...],
                              preferred_element_type=jnp.float32)

    slot = jax.lax.rem(t, _NBUF)
    pltpu.make_async_copy(
        adj_ref.at[pl.ds(jax.lax.rem(t, nb) * _BM, _BM), :],
        abuf.at[slot],
        sem.at[slot],
    ).wait()
    a = abuf[slot]

    @pl.when(p == 0)
    def _layer1():
        h = jnp.dot(a, s1_ref[...], preferred_element_type=jnp.float32)
        h = jnp.maximum(h, 0.0)
        s2 = jnp.dot(h, w2_ref[...], preferred_element_type=jnp.float32)
        s2_ref[pl.ds(b * _BM, _BM), :] = s2

    @pl.when(p == 1)
    def _layer2():
        out_ref[...] = jnp.dot(a, s2_ref[...],
                               preferred_element_type=jnp.float32)

    @pl.when(t + _NBUF < 2 * nb)
    def _refill():
        _start(t + _NBUF)


@jax.jit
def kernel(x, adj, W1, W2):
    n, nfeat = x.shape
    nhid = W1.shape[1]
    nout = W2.shape[1]
    nb = n // _BM

    return pl.pallas_call(
        functools.partial(_gcn_kernel, nb=nb),
        grid=(2, nb),
        in_specs=[
            pl.BlockSpec((n, nfeat), lambda p, b: (0, 0)),
            pl.BlockSpec((nfeat, nhid), lambda p, b: (0, 0)),
            pl.BlockSpec((nhid, nout), lambda p, b: (0, 0)),
            pl.BlockSpec(memory_space=pltpu.MemorySpace.HBM),
        ],
        out_specs=pl.BlockSpec((_BM, nout), lambda p, b: (b, 0)),
        out_shape=jax.ShapeDtypeStruct((n, nout), jnp.float32),
        scratch_shapes=[
            pltpu.VMEM((_NBUF, _BM, n), jnp.float32),
            pltpu.VMEM((n, nhid), jnp.float32),
            pltpu.VMEM((n, nout), jnp.float32),
            pltpu.SemaphoreType.DMA((_NBUF,)),
        ],
        compiler_params=pltpu.CompilerParams(
            vmem_limit_bytes=100 * 1024 * 1024,
        ),
    )(x, W1, W2, adj)


# speedup vs baseline: 1.0125x; 1.0125x over previous
import jax
import jax.numpy as jnp
from jax.experimental import pallas as pl
from jax.experimental.pallas import tpu as pltpu

_BM = 400


def _gcn_kernel(x_ref, w1_ref, w2_ref, adj_ref, out_ref, s1_ref, s2_ref):
    p = pl.program_id(0)
    b = pl.program_id(1)

    @pl.when(jnp.logical_and(p == 0, b == 0))
    def _prologue():
        s1_ref[...] = jnp.dot(x_ref[...], w1_ref[...],
                              preferred_element_type=jnp.float32)

    @pl.when(p == 0)
    def _layer1():
        h = jnp.dot(adj_ref[...], s1_ref[...],
                    preferred_element_type=jnp.float32)
        h = jnp.maximum(h, 0.0)
        s2 = jnp.dot(h, w2_ref[...], preferred_element_type=jnp.float32)
        s2_ref[pl.ds(b * _BM, _BM), :] = s2

    @pl.when(p == 1)
    def _layer2():
        out_ref[...] = jnp.dot(adj_ref[...], s2_ref[...],
                               preferred_element_type=jnp.float32)


@jax.jit
def kernel(x, adj, W1, W2):
    n, nfeat = x.shape
    nhid = W1.shape[1]
    nout = W2.shape[1]
    nb = n // _BM

    return pl.pallas_call(
        _gcn_kernel,
        grid=(2, nb),
        in_specs=[
            pl.BlockSpec((n, nfeat), lambda p, b: (0, 0)),
            pl.BlockSpec((nfeat, nhid), lambda p, b: (0, 0)),
            pl.BlockSpec((nhid, nout), lambda p, b: (0, 0)),
            pl.BlockSpec((_BM, n), lambda p, b: (b, 0)),
        ],
        out_specs=pl.BlockSpec((_BM, nout), lambda p, b: (b, 0)),
        out_shape=jax.ShapeDtypeStruct((n, nout), jnp.float32),
        scratch_shapes=[
            pltpu.VMEM((n, nhid), jnp.float32),
            pltpu.VMEM((n, nout), jnp.float32),
        ],
        compiler_params=pltpu.CompilerParams(
            vmem_limit_bytes=100 * 1024 * 1024,
        ),
    )(x, W1, W2, adj)
